# Initial kernel scaffold; baseline (speedup 1.0000x reference)
import jax
import jax.numpy as jnp
from jax.experimental import pallas as pl


def kernel(x1, edge_index1, edge_attr1, x2, edge_index2, edge_attr2, W1, b1, W2, b2, Wc1, bc1, Wc2, bc2, Wc3, bc3):
    raise NotImplementedError("write your pallas kernel here")


# speedup vs baseline: 33.5283x; 33.5283x over previous
import functools

import jax
import jax.numpy as jnp
from jax import lax
from jax.experimental import pallas as pl
from jax.experimental.pallas import tpu as pltpu
from jax.experimental.pallas import tpu_sc as plsc

_N = 200
_NPAD = 256
_E = 20000
_NC = 2
_NS = 16
_J = 5
_EPT = _J * 128
_M2 = _NPAD * _NPAD


def _sc_build_dense(src_a, dst_a, val_a):
    mesh = plsc.VectorSubcoreMesh(core_axis_name="c", subcore_axis_name="s")
    chunk = _M2 // _NS

    @functools.partial(
        pl.kernel,
        mesh=mesh,
        out_type=jax.ShapeDtypeStruct((_NC, 2, 2, _M2), jnp.float32),
        scratch_types=[
            pltpu.VMEM_SHARED((_M2,), jnp.float32),
            pltpu.VMEM_SHARED((_M2,), jnp.float32),
            pltpu.VMEM_SHARED((_M2,), jnp.float32),
            pltpu.VMEM_SHARED((_M2,), jnp.float32),
            pltpu.VMEM((_J, 128), jnp.int32),
            pltpu.VMEM((_J, 128), jnp.int32),
            pltpu.VMEM((_J, 128), jnp.float32),
            pltpu.VMEM((_J, 128), jnp.int32),
            pltpu.VMEM((_J, 128), jnp.int32),
            pltpu.VMEM((chunk,), jnp.float32),
        ],
    )
    def k(src_hbm, dst_hbm, val_hbm, out_hbm,
          sh_s0, sh_t0, sh_s1, sh_t1,
          src_v, dst_v, val_v, idx_s_v, idx_t_v, zeros_v):
        c = lax.axis_index("c")
        s = lax.axis_index("s")
        shared = (sh_s0, sh_t0, sh_s1, sh_t1)

        def zfill(i, carry):
            zeros_v[pl.ds(i * 16, 16)] = jnp.zeros((16,), jnp.float32)
            return carry

        lax.fori_loop(0, chunk // 16, zfill, 0)
        for r in shared:
            pltpu.sync_copy(zeros_v, r.at[pl.ds(s * chunk, chunk)])
        plsc.subcore_barrier()

        for g in range(2):
            pltpu.sync_copy(src_hbm.at[g, c, s], src_v)
            pltpu.sync_copy(dst_hbm.at[g, c, s], dst_v)
            pltpu.sync_copy(val_hbm.at[g, c, s], val_v)
            for j in range(_J):
                for i in range(8):
                    sl = pl.ds(i * 16, 16)
                    sv = src_v[j, sl]
                    dv = dst_v[j, sl]
                    idx_s_v[j, sl] = dv * _NPAD + sv
                    idx_t_v[j, sl] = sv * _NPAD + dv
            sh_s = shared[2 * g]
            sh_t = shared[2 * g + 1]
            for j in range(_J):
                pltpu.sync_copy(val_v.at[j], sh_s.at[idx_s_v.at[j]], add=True)
                pltpu.sync_copy(val_v.at[j], sh_t.at[idx_t_v.at[j]], add=True)
        plsc.subcore_barrier()

        for g in range(2):
            for m in range(2):
                r = shared[2 * g + m]
                pltpu.sync_copy(r.at[pl.ds(s * chunk, chunk)],
                                out_hbm.at[c, g, m, pl.ds(s * chunk, chunk)])

    return k(src_a, dst_a, val_a)


def _tc_body(s_ref, x1_ref, x2_ref, w1_ref, b1_ref, w2_ref, b2_ref,
             wc1_ref, bc1_ref, wc2_ref, bc2_ref, wc3_ref, bc3_ref,
             o1_ref, o2_ref):
    def mm(a, b):
        return jnp.dot(a, b, preferred_element_type=jnp.float32)

    def mm_hi(a, b):
        return jnp.dot(a, b, preferred_element_type=jnp.float32,
                       precision=lax.Precision.HIGHEST)

    def cheb(a_mat, x, w_ref, brow):
        out = mm(x, w_ref[0])
        tx1 = mm_hi(a_mat, x)
        out = out + mm(tx1, w_ref[1])
        tx2 = 2.0 * mm_hi(a_mat, tx1) - x
        out = out + mm(tx2, w_ref[2])
        return out + brow

    for g, (x_ref, o_ref) in enumerate(((x1_ref, o1_ref), (x2_ref, o2_ref))):
        s_mat = s_ref[0, g, 0] + s_ref[1, g, 0]
        st_mat = s_ref[0, g, 1] + s_ref[1, g, 1]
        deg_row = jnp.sum(s_mat, axis=0, keepdims=True)
        deg_col = jnp.sum(st_mat, axis=1, keepdims=True)
        dinv_row = jnp.where(deg_row > 0,
                             1.0 / jnp.sqrt(jnp.where(deg_row > 0, deg_row, 1.0)), 0.0)
        dinv_col = jnp.where(deg_col > 0,
                             1.0 / jnp.sqrt(jnp.where(deg_col > 0, deg_col, 1.0)), 0.0)
        a_mat = -(dinv_col * s_mat * dinv_row)
        x = x_ref[...]
        h = jnp.maximum(cheb(a_mat, x, w1_ref, b1_ref[...]), 0.0)
        o = cheb(a_mat, h, w2_ref, b2_ref[...])
        z = lax.dot_general(o, wc1_ref[...], (((0,), (0,)), ((), ())),
                            preferred_element_type=jnp.float32)
        h1 = jnp.maximum(z + bc1_ref[...], 0.0)
        h2 = jnp.maximum(mm(h1, wc2_ref[...]) + bc2_ref[...], 0.0)
        res = mm(h2, wc3_ref[...]) + bc3_ref[...]
        o_ref[...] = res[0:8, :]


def _chunk_i32(a, pad):
    a = jnp.concatenate([a.astype(jnp.int32), jnp.zeros((pad,), jnp.int32)])
    return a.reshape(_NC, _NS, _J, 128)


def _chunk_f32(a, pad):
    a = jnp.concatenate([a.astype(jnp.float32), jnp.zeros((pad,), jnp.float32)])
    return a.reshape(_NC, _NS, _J, 128)


def kernel(x1, edge_index1, edge_attr1, x2, edge_index2, edge_attr2,
           W1, b1, W2, b2, Wc1, bc1, Wc2, bc2, Wc3, bc3):
    pad = _NC * _NS * _EPT - _E
    src_a = jnp.stack([_chunk_i32(edge_index1[0], pad),
                       _chunk_i32(edge_index2[0], pad)])
    dst_a = jnp.stack([_chunk_i32(edge_index1[1], pad),
                       _chunk_i32(edge_index2[1], pad)])
    val_a = jnp.stack([_chunk_f32(edge_attr1, pad),
                       _chunk_f32(edge_attr2, pad)])

    s_flat = _sc_build_dense(src_a, dst_a, val_a)
    s_all = s_flat.reshape(_NC, 2, 2, _NPAD, _NPAD)

    rpad = _NPAD - _N
    x1p = jnp.pad(x1, ((0, rpad), (0, 0)))
    x2p = jnp.pad(x2, ((0, rpad), (0, 0)))
    b1r = b1.reshape(1, -1)
    w2p = jnp.pad(W2, ((0, 0), (0, 0), (0, 126)))
    b2r = jnp.pad(b2, (0, 126)).reshape(1, 128)
    wc1p = jnp.pad(Wc1, ((0, 56), (0, 28)))
    bc1r = jnp.pad(bc1, (0, 28)).reshape(1, 128)
    wc2p = jnp.pad(Wc2, ((0, 28), (0, 78)))
    bc2r = jnp.pad(bc2, (0, 78)).reshape(1, 128)
    wc3p = jnp.pad(Wc3, ((0, 78), (0, 127)))
    bc3r = jnp.pad(bc3, (0, 127)).reshape(1, 128)

    o1, o2 = pl.pallas_call(
        _tc_body,
        out_shape=[jax.ShapeDtypeStruct((8, 128), jnp.float32),
                   jax.ShapeDtypeStruct((8, 128), jnp.float32)],
    )(s_all, x1p, x2p, W1, b1r, w2p, b2r,
      wc1p, bc1r, wc2p, bc2r, wc3p, bc3r)
    return (o1[:2, :1], o2[:2, :1])
